# Initial kernel scaffold; baseline (speedup 1.0000x reference)
import jax
import jax.numpy as jnp
from jax.experimental import pallas as pl


def kernel(node_features, latent_features, rigids7, edge_features, seq_edge_features, edge_index, seq_edge_index, res_mask, noising_mask, params):
    raise NotImplementedError("write your pallas kernel here")


# speedup vs baseline: 1.0074x; 1.0074x over previous
import jax
import jax.numpy as jnp
import numpy as np
from jax.experimental import pallas as pl
from jax.experimental.pallas import tpu as pltpu

_N = 10000
_E = 320000
_ES = 60000
_CS = 256
_CL = 128
_CZ = 128
_EDGE_IN = 128
_H = 8
_D = 32
_PQK = 8
_PV = 12


def _ln(x, g, b):
    m = x.mean(-1, keepdims=True)
    v = x.var(-1, keepdims=True)
    return (x - m) / jnp.sqrt(v + 1e-5) * g + b


def _quat_to_rot(q):
    w, x, y, z = q[:, 0], q[:, 1], q[:, 2], q[:, 3]
    R = jnp.stack([
        1 - 2 * (y * y + z * z), 2 * (x * y - w * z), 2 * (x * z + w * y),
        2 * (x * y + w * z), 1 - 2 * (x * x + z * z), 2 * (y * z - w * x),
        2 * (x * z - w * y), 2 * (y * z + w * x), 1 - 2 * (x * x + y * y)
    ], axis=-1).reshape(-1, 3, 3)
    return R


def _quat_mul(q, p):
    w1, x1, y1, z1 = q[:, 0], q[:, 1], q[:, 2], q[:, 3]
    w2, x2, y2, z2 = p[:, 0], p[:, 1], p[:, 2], p[:, 3]
    return jnp.stack([
        w1 * w2 - x1 * x2 - y1 * y2 - z1 * z2,
        w1 * x2 + x1 * w2 + y1 * z2 - z1 * y2,
        w1 * y2 - x1 * z2 + y1 * w2 + z1 * x2,
        w1 * z2 + x1 * y2 - y1 * x2 + z1 * w2
    ], axis=-1)


def _edge_mlp_body(x_ref, w0_ref, b0_ref, w1_ref, b1_ref, w2_ref, b2_ref,
                   g_ref, bg_ref, wb_ref, z_ref, bb_ref):
    x = x_ref[...]
    h = jnp.maximum(jnp.dot(x, w0_ref[...], preferred_element_type=jnp.float32)
                    + b0_ref[...], 0.0)
    h = jnp.maximum(jnp.dot(h, w1_ref[...], preferred_element_type=jnp.float32)
                    + b1_ref[...], 0.0)
    z = jnp.dot(h, w2_ref[...], preferred_element_type=jnp.float32) + b2_ref[...]
    m = z.mean(-1, keepdims=True)
    v = ((z - m) ** 2).mean(-1, keepdims=True)
    zn = (z - m) / jnp.sqrt(v + 1e-5) * g_ref[...] + bg_ref[...]
    z_ref[...] = zn
    bb_ref[...] = jnp.dot(zn, wb_ref[...], preferred_element_type=jnp.float32)


def _edge_mlp(ef, p):
    e = ef.shape[0]
    be = 1600
    nblk = e // be
    full = lambda shape: pl.BlockSpec(shape, lambda i: (0, 0))
    z, bb = pl.pallas_call(
        _edge_mlp_body,
        grid=(nblk,),
        in_specs=[
            pl.BlockSpec((be, _EDGE_IN), lambda i: (i, 0)),
            full((_EDGE_IN, _CZ)), full((1, _CZ)),
            full((_CZ, _CZ)), full((1, _CZ)),
            full((_CZ, _CZ)), full((1, _CZ)),
            full((1, _CZ)), full((1, _CZ)),
            full((_CZ, _H)),
        ],
        out_specs=[
            pl.BlockSpec((be, _CZ), lambda i: (i, 0)),
            pl.BlockSpec((be, _H), lambda i: (i, 0)),
        ],
        out_shape=[
            jax.ShapeDtypeStruct((e, _CZ), jnp.float32),
            jax.ShapeDtypeStruct((e, _H), jnp.float32),
        ],
    )(ef, p['We0'], p['be0'].reshape(1, -1), p['We1'], p['be1'].reshape(1, -1),
      p['We2'], p['be2'].reshape(1, -1), p['ge'].reshape(1, -1),
      p['bge'].reshape(1, -1), p['sp_Wb'])
    return z, bb


def _ipa(s, z, b, ei, R, t, mask, p, pre):
    n = s.shape[0]
    src, dst = ei[0], ei[1]
    q = (s @ p[pre + 'Wq']).reshape(n, _H, _D)
    k = (s @ p[pre + 'Wk']).reshape(n, _H, _D)
    v = (s @ p[pre + 'Wv']).reshape(n, _H, _D)
    qp = (s @ p[pre + 'Wqp']).reshape(n, _H, _PQK, 3)
    kp = (s @ p[pre + 'Wkp']).reshape(n, _H, _PQK, 3)
    vp = (s @ p[pre + 'Wvp']).reshape(n, _H, _PV, 3)
    qp = jnp.einsum('nij,nhpj->nhpi', R, qp) + t[:, None, None, :]
    kp = jnp.einsum('nij,nhpj->nhpi', R, kp) + t[:, None, None, :]
    vp = jnp.einsum('nij,nhpj->nhpi', R, vp) + t[:, None, None, :]
    qk = jnp.sum(q[dst] * k[src], -1) / jnp.sqrt(3.0 * _D)
    d2 = jnp.sum((qp[dst] - kp[src]) ** 2, axis=(-2, -1))
    hw = jax.nn.softplus(p[pre + 'head_w'])
    pt = -0.5 * d2 * hw * jnp.sqrt(1.0 / (3.0 * _PQK * 4.5))
    a = qk + b / jnp.sqrt(3.0) + pt + (mask[src] - 1.0)[:, None] * 1e9
    amax = jax.ops.segment_max(a, dst, num_segments=n)
    amax = jnp.where(jnp.isfinite(amax), amax, 0.0)
    ea = jnp.exp(a - amax[dst])
    den = jax.ops.segment_sum(ea, dst, num_segments=n)
    attn = ea / (den[dst] + 1e-9)
    o = jax.ops.segment_sum(attn[:, :, None] * v[src], dst, num_segments=n)
    opt = jax.ops.segment_sum(attn[:, :, None, None] * vp[src], dst, num_segments=n)
    optl = jnp.einsum('nji,nhpj->nhpi', R, opt - t[:, None, None, :])
    onorm = jnp.sqrt(jnp.sum(optl ** 2, -1) + 1e-8)
    opair = jax.ops.segment_sum(attn[:, :, None] * z[:, None, :], dst, num_segments=n)
    cat = jnp.concatenate([o.reshape(n, -1), optl.reshape(n, -1),
                           onorm.reshape(n, -1), opair.reshape(n, -1)], -1)
    return cat @ p[pre + 'Wo'] + p[pre + 'bo']


def kernel(node_features, latent_features, rigids7, edge_features,
           seq_edge_features, edge_index, seq_edge_index, res_mask,
           noising_mask, params):
    p = params
    nf, lf, rg, ef, sef = (node_features, latent_features, rigids7,
                           edge_features, seq_edge_features)
    ei, sei, rmask, nmask = edge_index, seq_edge_index, res_mask, noising_mask

    qq = rg[:, :4]
    qq = qq / jnp.linalg.norm(qq, axis=-1, keepdims=True)
    t = rg[:, 4:]
    R = _quat_to_rot(qq)

    z, bb = _edge_mlp(ef, p)
    bb2 = sef @ p['sq_Wb']

    s = jnp.concatenate([nf, lf], -1) @ p['Wnu'] + p['bnu']
    u = _ipa(s, z, bb, ei, R, t, rmask, p, 'sp_') * rmask[:, None]
    s = _ln(s + u, p['g1'], p['b1'])
    u = _ipa(s, sef, bb2, sei, R, t, rmask, p, 'sq_') * rmask[:, None]
    s = _ln(s + u, p['g1'], p['b1'])
    x = jax.nn.relu(s @ p['Wt1'] + p['bt1'])
    x = jax.nn.relu(x @ p['Wt2'] + p['bt2'])
    x = x @ p['Wt3'] + p['bt3']
    s = _ln(s + x, p['gt'], p['bgt'])
    s = s * rmask[:, None]
    ub = (s * nmask[:, None]) @ p['Wbb'] + p['bbb']
    ub = ub * nmask[:, None]
    qu = jnp.concatenate([jnp.ones((s.shape[0], 1), dtype=s.dtype), ub[:, :3]], -1)
    qn = _quat_mul(qq, qu)
    qn = qn / jnp.linalg.norm(qn, axis=-1, keepdims=True)
    tn = t + jnp.einsum('nij,nj->ni', R, ub[:, 3:])
    lt = lf + s @ p['Wlu'] + p['blu']
    return s, jnp.concatenate([qn, tn], -1), lt


# speedup vs baseline: 11.7229x; 11.6368x over previous
import functools
import numpy as np

import jax
import jax.numpy as jnp
from jax import lax
from jax.experimental import pallas as pl
from jax.experimental.pallas import tpu as pltpu
from jax.experimental.pallas import tpu_sc as plsc

H = 8
D = 32
PQK = 8
PV = 12
CZ = 128

NR = 16
NW = 32
FC = 8

RSQK = 1.0 / np.sqrt(3.0 * D)
RS3 = 1.0 / np.sqrt(3.0)
CPT = 0.5 * np.sqrt(1.0 / (3.0 * PQK * 4.5))

ACC_W = 1696
O_OFF = 16
OPT_OFF = 272
OPAIR_OFF = 656
SV_OFF = 512
SVP_OFF = 768
SRC_W = 1152
ET_W = 256

@functools.lru_cache(maxsize=1)
def _sc_mesh():
    return plsc.VectorSubcoreMesh(core_axis_name="c", subcore_axis_name="s",
                                  num_cores=2, num_subcores=16)


def _wid():
    return lax.axis_index("s") * 2 + lax.axis_index("c")


def _pick_cb(ew):
    for cb in range(128, 7, -8):
        if ew % cb == 0:
            return cb
    raise ValueError(ew)


def _vgather(vec, idx):
    dn = lax.GatherDimensionNumbers(offset_dims=(), collapsed_slice_dims=(0,),
                                    start_index_map=(0,))
    return lax.gather(vec, idx[:, None], dimension_numbers=dn, slice_sizes=(1,),
                      mode=lax.GatherScatterMode.PROMISE_IN_BOUNDS)


def _frames(rg):
    qq = rg[:, 0:4]
    nrm = jnp.sqrt(jnp.sum(qq * qq, -1, keepdims=True))
    qqn = qq / nrm
    w, x, y, z = qqn[:, 0:1], qqn[:, 1:2], qqn[:, 2:3], qqn[:, 3:4]
    R = ((1 - 2 * (y * y + z * z), 2 * (x * y - w * z), 2 * (x * z + w * y)),
        (2 * (x * y + w * z), 1 - 2 * (x * x + z * z), 2 * (y * z - w * x)),
        (2 * (x * z - w * y), 2 * (y * z + w * x), 1 - 2 * (x * x + y * y)))
    t = (rg[:, 4:5], rg[:, 5:6], rg[:, 6:7])
    return R, t, qqn


def _edge_mlp_body(x_ref, w0_ref, b0_ref, w1_ref, b1_ref, w2_ref, b2_ref,
                   g_ref, bg_ref, wb_ref, out_ref):
    x = x_ref[...]
    h = jnp.maximum(jnp.dot(x, w0_ref[...], preferred_element_type=jnp.float32)
                    + b0_ref[...], 0.0)
    h = jnp.maximum(jnp.dot(h, w1_ref[...], preferred_element_type=jnp.float32)
                    + b1_ref[...], 0.0)
    zz = jnp.dot(h, w2_ref[...], preferred_element_type=jnp.float32) + b2_ref[...]
    m = zz.mean(-1, keepdims=True)
    v = ((zz - m) ** 2).mean(-1, keepdims=True)
    zn = (zz - m) / jnp.sqrt(v + 1e-5) * g_ref[...] + bg_ref[...]
    out_ref[:, 0:CZ] = zn
    out_ref[:, CZ:CZ + H] = jnp.dot(zn, wb_ref[...],
                                    preferred_element_type=jnp.float32)
    out_ref[:, CZ + H:ET_W] = jnp.zeros_like(out_ref[:, CZ + H:ET_W])


def _edge_mlp(ef, p):
    e = ef.shape[0]
    be = 1600 if e % 1600 == 0 else 128
    nblk = e // be
    full = lambda shape: pl.BlockSpec(shape, lambda i: (0, 0))
    inrow = pl.BlockSpec((be, CZ), lambda i: (jnp.minimum(i, nblk - 1), 0))
    return pl.pallas_call(
        _edge_mlp_body,
        grid=(nblk + 1,),
        in_specs=[inrow, full((CZ, CZ)), full((1, CZ)), full((CZ, CZ)),
                  full((1, CZ)), full((CZ, CZ)), full((1, CZ)), full((1, CZ)),
                  full((1, CZ)), full((CZ, H))],
        out_specs=pl.BlockSpec((be, ET_W), lambda i: (i, 0)),
        out_shape=jax.ShapeDtypeStruct(((nblk + 1) * be, ET_W), jnp.float32),
    )(ef, p['We0'], p['be0'].reshape(1, -1), p['We1'], p['be1'].reshape(1, -1),
      p['We2'], p['be2'].reshape(1, -1), p['ge'].reshape(1, -1),
      p['bge'].reshape(1, -1), p['sp_Wb'])


def _edge_tab2_body(x_ref, wb_ref, out_ref):
    x = x_ref[...]
    out_ref[:, 0:CZ] = x
    out_ref[:, CZ:CZ + H] = jnp.dot(x, wb_ref[...],
                                    preferred_element_type=jnp.float32)
    out_ref[:, CZ + H:ET_W] = jnp.zeros_like(out_ref[:, CZ + H:ET_W])


def _edge_tab2(sefp, p):
    e = sefp.shape[0]
    be = 1280 if e % 1280 == 0 else 128
    nblk = e // be
    return pl.pallas_call(
        _edge_tab2_body,
        grid=(nblk + 1,),
        in_specs=[pl.BlockSpec((be, CZ), lambda i: (jnp.minimum(i, nblk - 1), 0)),
                  pl.BlockSpec((CZ, H), lambda i: (0, 0))],
        out_specs=pl.BlockSpec((be, ET_W), lambda i: (i, 0)),
        out_shape=jax.ShapeDtypeStruct(((nblk + 1) * be, ET_W), jnp.float32),
    )(sefp, p['sq_Wb'])


def _s0_body(x_ref, w_ref, b_ref, o_ref):
    o_ref[...] = jnp.dot(x_ref[...], w_ref[...],
                         preferred_element_type=jnp.float32) + b_ref[...]


def _s0(nflf, p):
    n = nflf.shape[0]
    bn = 1000 if n % 1000 == 0 else 16
    cin = nflf.shape[1]
    return pl.pallas_call(
        _s0_body,
        grid=(n // bn,),
        in_specs=[pl.BlockSpec((bn, cin), lambda i: (i, 0)),
                  pl.BlockSpec((cin, 256), lambda i: (0, 0)),
                  pl.BlockSpec((1, 256), lambda i: (0, 0))],
        out_specs=pl.BlockSpec((bn, 256), lambda i: (i, 0)),
        out_shape=jax.ShapeDtypeStruct((n, 256), jnp.float32),
    )(nflf, p['Wnu'], p['bnu'].reshape(1, -1))


def _tab_body(s_ref, rg_ref, wq_ref, wk_ref, wv_ref, wqp_ref, wkp_ref, wvp_ref,
              hw_ref, dst_ref, src_ref, hwc_ref):
    s = s_ref[...]
    R, t, _ = _frames(rg_ref[...])
    q = jnp.dot(s, wq_ref[...], preferred_element_type=jnp.float32)
    k = jnp.dot(s, wk_ref[...], preferred_element_type=jnp.float32)
    v = jnp.dot(s, wv_ref[...], preferred_element_type=jnp.float32)
    qp3 = jnp.dot(s, wqp_ref[...], preferred_element_type=jnp.float32)
    kp3 = jnp.dot(s, wkp_ref[...], preferred_element_type=jnp.float32)
    vp3 = jnp.dot(s, wvp_ref[...], preferred_element_type=jnp.float32)

    def rot(p3, width):
        px = p3[:, 0 * width:1 * width]
        py = p3[:, 1 * width:2 * width]
        pz = p3[:, 2 * width:3 * width]
        return tuple(R[i][0] * px + R[i][1] * py + R[i][2] * pz + t[i]
                     for i in range(3))

    qpg = rot(qp3, H * PQK)
    kpg = rot(kp3, H * PQK)
    vpg = rot(vp3, H * PV)
    zpad = jnp.zeros_like(s[:, 0:8])
    for h in range(H):
        b = h * 64
        dst_ref[:, b:b + 32] = q[:, h * 32:h * 32 + 32]
        src_ref[:, b:b + 32] = k[:, h * 32:h * 32 + 32]
        for c in range(3):
            dst_ref[:, b + 32 + c * 8:b + 40 + c * 8] = qpg[c][:, h * 8:h * 8 + 8]
            src_ref[:, b + 32 + c * 8:b + 40 + c * 8] = kpg[c][:, h * 8:h * 8 + 8]
        dst_ref[:, b + 56:b + 64] = zpad
        src_ref[:, b + 56:b + 64] = zpad
    src_ref[:, SV_OFF:SV_OFF + 256] = v
    zpad4 = jnp.zeros_like(s[:, 0:4])
    for c in range(3):
        for h in range(H):
            b0 = SVP_OFF + c * 128 + h * 16
            src_ref[:, b0:b0 + 12] = vpg[c][:, h * 12:(h + 1) * 12]
            src_ref[:, b0 + 12:b0 + 16] = zpad4
    hw = hw_ref[...]
    ch = jnp.log1p(jnp.exp(hw)) * CPT
    hwc_ref[...] = jnp.concatenate(
        [ch, jnp.zeros((1, 128 - H), jnp.float32)], axis=1)


def _tables(s, rg, p, pre):
    n = s.shape[0]
    bn = 1000 if n % 1000 == 0 else 16
    full = lambda shape: pl.BlockSpec(shape, lambda i: (0, 0))
    return pl.pallas_call(
        _tab_body,
        grid=(n // bn,),
        in_specs=[pl.BlockSpec((bn, 256), lambda i: (i, 0)),
                  pl.BlockSpec((bn, 7), lambda i: (i, 0)),
                  full((256, 256)), full((256, 256)), full((256, 256)),
                  full((256, 192)), full((256, 192)), full((256, 288)),
                  full((1, H))],
        out_specs=[pl.BlockSpec((bn, 512), lambda i: (i, 0)),
                   pl.BlockSpec((bn, SRC_W), lambda i: (i, 0)),
                   pl.BlockSpec((1, 128), lambda i: (0, 0))],
        out_shape=[jax.ShapeDtypeStruct((n, 512), jnp.float32),
                   jax.ShapeDtypeStruct((n, SRC_W), jnp.float32),
                   jax.ShapeDtypeStruct((1, 128), jnp.float32)],
    )(s, rg, p[pre + 'Wq'], p[pre + 'Wk'], p[pre + 'Wv'], p[pre + 'Wqp_p'],
      p[pre + 'Wkp_p'], p[pre + 'Wvp_p'], p[pre + 'head_w'].reshape(1, -1))


def _hist(dstx, e2, nbp):
    ew = e2 // NW
    cb = _pick_cb(ew)

    def body(dst_hbm, hist_hbm, dbuf, histv, i32z16):
        w = _wid()
        zv = jnp.zeros((16,), jnp.int32)
        for j in range(nbp // 16):
            histv[pl.ds(j * 16, 16)] = zv

        def chunk(c, _):
            pltpu.sync_copy(dst_hbm.at[pl.ds(w * ew + c * cb, cb)], dbuf)

            def per(j, _):
                b = dbuf[pl.ds(j, 1)][0] >> 4
                old = histv[pl.ds(b, 1)][0]
                histv[pl.ds(b, 1)] = jnp.full((1,), old + 1, jnp.int32)
                return 0
            lax.fori_loop(0, cb, per, 0)
            return 0
        lax.fori_loop(0, ew // cb, chunk, 0)
        pltpu.sync_copy(histv, hist_hbm.at[w])

    return pl.kernel(
        body, out_type=jax.ShapeDtypeStruct((NW, nbp), jnp.int32),
        mesh=_sc_mesh(),
        scratch_types=[pltpu.VMEM((cb,), jnp.int32),
                       pltpu.VMEM((nbp,), jnp.int32),
                       pltpu.VMEM((16,), jnp.int32)],
    )(dstx)


def _prefix_body(h_ref, start_ref, off_ref, pc_ref, cnt_ref):
    h = h_ref[...].astype(jnp.float32)
    nbp = h.shape[1]
    wi = lax.broadcasted_iota(jnp.int32, (NW, NW), 0)
    wj = lax.broadcasted_iota(jnp.int32, (NW, NW), 1)
    mlow = (wj < wi).astype(jnp.float32)
    below = jnp.dot(mlow, h, preferred_element_type=jnp.float32)
    cnt = jnp.sum(h, axis=0, keepdims=True)
    pc = jnp.floor((cnt + 31.0) * (1.0 / 32.0)) * 32.0
    bi = lax.broadcasted_iota(jnp.int32, (nbp, nbp), 0)
    bj = lax.broadcasted_iota(jnp.int32, (nbp, nbp), 1)
    mb = (bi < bj).astype(jnp.float32)
    off = jnp.dot(pc, mb, preferred_element_type=jnp.float32)
    start_ref[...] = jnp.round(below + off).astype(jnp.int32)
    ones8 = jnp.ones((8, 1), jnp.float32)
    off_ref[...] = jnp.round(ones8 * off).astype(jnp.int32)
    pc_ref[...] = jnp.round(ones8 * pc).astype(jnp.int32)
    cnt_ref[...] = jnp.round(ones8 * cnt).astype(jnp.int32)


def _prefix(hist, nbp):
    full = lambda shape: pl.BlockSpec(shape, lambda: (0, 0))
    return pl.pallas_call(
        _prefix_body,
        in_specs=[full((NW, nbp))],
        out_specs=[full((NW, nbp)), full((8, nbp)), full((8, nbp)),
                   full((8, nbp))],
        out_shape=[jax.ShapeDtypeStruct((NW, nbp), jnp.int32),
                   jax.ShapeDtypeStruct((8, nbp), jnp.int32),
                   jax.ShapeDtypeStruct((8, nbp), jnp.int32),
                   jax.ShapeDtypeStruct((8, nbp), jnp.int32)],
    )(hist)


def _place(dstx, srcx, start, off, pc, cnt, e2, nb, nbp, lp, edummy):
    ew = e2 // NW
    cb = _pick_cb(ew)

    def body(dst_hbm, src_hbm, start_hbm, off_hbm, pc_hbm, cnt_hbm,
             bid_hbm, bsrc_hbm, bdst_hbm,
             dbuf, sbuf, curv, posb, idb, offv, pcv, cntv, padpos, sent, sem):
        w = _wid()
        pltpu.sync_copy(start_hbm.at[w], curv)
        pltpu.sync_copy(off_hbm.at[0], offv)
        pltpu.sync_copy(pc_hbm.at[0], pcv)
        pltpu.sync_copy(cnt_hbm.at[0], cntv)
        sent[0, pl.ds(0, 16)] = jnp.full((16,), edummy, jnp.int32)
        sent[1, pl.ds(0, 16)] = jnp.zeros((16,), jnp.int32)
        sent[2, pl.ds(0, 16)] = jnp.full((16,), -16, jnp.int32)

        def chunk(c, _):
            pltpu.sync_copy(dst_hbm.at[pl.ds(w * ew + c * cb, cb)], dbuf)
            pltpu.sync_copy(src_hbm.at[pl.ds(w * ew + c * cb, cb)], sbuf)

            def per(j, _):
                b = dbuf[pl.ds(j, 1)][0] >> 4
                pos = curv[pl.ds(b, 1)][0]
                curv[pl.ds(b, 1)] = jnp.full((1,), pos + 1, jnp.int32)
                posb[pl.ds(j, 1)] = jnp.full((1,), pos, jnp.int32)
                idb[pl.ds(j, 1)] = jnp.full((1,), w * ew + c * cb + j,
                                            jnp.int32)
                return 0
            lax.fori_loop(0, cb, per, 0)
            pltpu.async_copy(idb, bid_hbm.at[posb], sem).wait()
            pltpu.async_copy(sbuf, bsrc_hbm.at[posb], sem).wait()
            pltpu.async_copy(dbuf, bdst_hbm.at[posb], sem).wait()
            return 0
        lax.fori_loop(0, ew // cb, chunk, 0)

        trips = jnp.maximum((nb - w) // NW + 1, 0)

        def padfill(i, _):
            b = w + i * NW
            pcb = pcv[pl.ds(b, 1)][0]
            cntb = cntv[pl.ds(b, 1)][0]
            offb = offv[pl.ds(b, 1)][0]
            pad = pcb - cntb
            base = offb + cntb
            last = offb + pcb - 1

            @pl.when(pad > 0)
            def _():
                for r in range(2):
                    for l in range(16):
                        padpos[r, pl.ds(l, 1)] = jnp.full(
                            (1,), jnp.minimum(base + r * 16 + l, last),
                            jnp.int32)
                for r in range(2):
                    pltpu.async_copy(sent.at[0], bid_hbm.at[padpos.at[r]], sem).wait()
                    pltpu.async_copy(sent.at[1], bsrc_hbm.at[padpos.at[r]], sem).wait()
                    pltpu.async_copy(sent.at[2], bdst_hbm.at[padpos.at[r]], sem).wait()
            return 0
        lax.fori_loop(0, trips, padfill, 0)

    return pl.kernel(
        body,
        out_type=[jax.ShapeDtypeStruct((lp,), jnp.int32),
                  jax.ShapeDtypeStruct((lp,), jnp.int32),
                  jax.ShapeDtypeStruct((lp,), jnp.int32)],
        mesh=_sc_mesh(),
        scratch_types=[pltpu.VMEM((cb,), jnp.int32),
                       pltpu.VMEM((cb,), jnp.int32),
                       pltpu.VMEM((nbp,), jnp.int32),
                       pltpu.VMEM((cb,), jnp.int32),
                       pltpu.VMEM((cb,), jnp.int32),
                       pltpu.VMEM((nbp,), jnp.int32),
                       pltpu.VMEM((nbp,), jnp.int32),
                       pltpu.VMEM((nbp,), jnp.int32),
                       pltpu.VMEM((2, 16), jnp.int32),
                       pltpu.VMEM((3, 16), jnp.int32),
                       pltpu.SemaphoreType.DMA],
    )(dstx, srcx, start, off, pc, cnt)


def _attn(dtab, stab, etab, bid, bsrc, bdst, off2d, pc2d, hwc, eye16,
          n, nb, nbp):
    def body(dtab_hbm, stab_hbm, etab_hbm, bid_hbm, bsrc_hbm, bdst_hbm,
             off_hbm, pc_hbm, hwc_hbm, eye_hbm, acc_hbm,
             hwv, dtabv, accv, obuf, pbuf, srcv, idsv, dstv, srows, erows,
             ohv, sem):
        w = _wid()
        pltpu.sync_copy(hwc_hbm.at[0], hwv)
        pltpu.sync_copy(eye_hbm, ohv)
        onehots = [ohv[hh, pl.ds(0, 16)] for hh in range(H)]
        hv = hwv[pl.ds(0, 16)]
        z16 = jnp.zeros((16,), jnp.float32)
        trips = ((nb - 1 - w) >> 5) + 1

        def bucket(i, _):
            b = w + i * NW
            nb0 = b * NR

            def zrow(r, _2):
                for cix in range(ACC_W // 16):
                    accv[r, pl.ds(cix * 16, 16)] = z16
                return 0
            lax.fori_loop(0, NR + 1, zrow, 0)
            pltpu.sync_copy(dtab_hbm.at[pl.ds(nb0, NR)], dtabv)
            pltpu.sync_copy(off_hbm.at[b], obuf)
            pltpu.sync_copy(pc_hbm.at[b], pbuf)
            o0 = obuf[...][0]
            nchunks = pbuf[...][0] >> 3

            def chunk(ci, _2):
                base = pl.multiple_of(o0 + ci * FC, 8)
                pltpu.sync_copy(bid_hbm.at[pl.ds(base, FC)], idsv)
                pltpu.sync_copy(bsrc_hbm.at[pl.ds(base, FC)], srcv)
                pltpu.sync_copy(bdst_hbm.at[pl.ds(base, FC)],
                                dstv.at[pl.ds(0, FC)])
                pltpu.async_copy(stab_hbm.at[srcv], srows, sem).wait()
                pltpu.async_copy(etab_hbm.at[idsv], erows, sem).wait()
                dsts = dstv[...]
                for j in range(FC):
                    d = dsts[j]
                    dl = jnp.where((d >> 4) == b, d - nb0, NR)
                    dlr = jnp.minimum(dl, NR - 1)
                    bbv = erows[j, pl.ds(CZ, 16)]
                    avec = z16
                    for h in range(H):
                        cb0 = h * 64
                        qv0 = dtabv[dlr, pl.ds(cb0, 16)]
                        qv1 = dtabv[dlr, pl.ds(cb0 + 16, 16)]
                        qv2 = dtabv[dlr, pl.ds(cb0 + 32, 16)]
                        qv3 = dtabv[dlr, pl.ds(cb0 + 48, 16)]
                        kv0 = srows[j, pl.ds(cb0, 16)]
                        kv1 = srows[j, pl.ds(cb0 + 16, 16)]
                        kv2 = srows[j, pl.ds(cb0 + 32, 16)]
                        kv3 = srows[j, pl.ds(cb0 + 48, 16)]
                        qk = jnp.sum(qv0 * kv0 + qv1 * kv1)
                        dq2 = qv2 - kv2
                        dq3 = qv3 - kv3
                        d2 = jnp.sum(dq2 * dq2 + dq3 * dq3)
                        a = qk * RSQK + bbv[h] * RS3 - hv[h] * d2
                        avec = avec + a * onehots[h]
                    vea = jnp.exp(avec)
                    accv[dl, pl.ds(0, 16)] = accv[dl, pl.ds(0, 16)] + vea
                    zv = [erows[j, pl.ds(r * 16, 16)] for r in range(8)]
                    for h in range(H):
                        eb = jnp.full((16,), vea[h], jnp.float32)
                        for r in range(2):
                            co = O_OFF + h * 32 + r * 16
                            si = SV_OFF + h * 32 + r * 16
                            accv[dl, pl.ds(co, 16)] = (
                                accv[dl, pl.ds(co, 16)]
                                + eb * srows[j, pl.ds(si, 16)])
                        for c in range(3):
                            co = OPT_OFF + c * 128 + h * 16
                            si = SVP_OFF + c * 128 + h * 16
                            accv[dl, pl.ds(co, 16)] = (
                                accv[dl, pl.ds(co, 16)]
                                + eb * srows[j, pl.ds(si, 16)])
                        for r in range(8):
                            cp = OPAIR_OFF + h * 128 + r * 16
                            accv[dl, pl.ds(cp, 16)] = (
                                accv[dl, pl.ds(cp, 16)] + eb * zv[r])
                return 0
            lax.fori_loop(0, nchunks, chunk, 0)
            pltpu.sync_copy(accv.at[pl.ds(0, NR)], acc_hbm.at[pl.ds(nb0, NR)])
            return 0
        lax.fori_loop(0, trips, bucket, 0)

    return pl.kernel(
        body, out_type=jax.ShapeDtypeStruct((n, ACC_W), jnp.float32),
        mesh=_sc_mesh(),
        compiler_params=pltpu.CompilerParams(needs_layout_passes=False),
        scratch_types=[pltpu.VMEM((128,), jnp.float32),
                       pltpu.VMEM((NR, 512), jnp.float32),
                       pltpu.VMEM((NR + 1, ACC_W), jnp.float32),
                       pltpu.VMEM((16,), jnp.int32),
                       pltpu.VMEM((16,), jnp.int32),
                       pltpu.VMEM((FC,), jnp.int32),
                       pltpu.VMEM((FC,), jnp.int32),
                       pltpu.VMEM((16,), jnp.int32),
                       pltpu.VMEM((FC, SRC_W), jnp.float32),
                       pltpu.VMEM((FC, ET_W), jnp.float32),
                       pltpu.VMEM((16, 16), jnp.float32),
                       pltpu.SemaphoreType.DMA],
    )(dtab, stab, etab, bid, bsrc, bdst, off2d, pc2d, hwc, eye16)


def _post_body(acc_ref, s_ref, rg_ref, wo_ref, bo_ref, g_ref, b_ref, rm_ref,
               out_ref):
    acc = acc_ref[...]
    R, t, _ = _frames(rg_ref[...])
    den = acc[:, 0:H]
    inv = 1.0 / (den + 1e-9)
    parts = []
    for h in range(H):
        parts.append(acc[:, O_OFF + h * 32:O_OFF + (h + 1) * 32]
                     * inv[:, h:h + 1])
    optn = []
    for c in range(3):
        blk = []
        for h in range(H):
            b0 = OPT_OFF + c * 128 + h * 16
            blk.append(acc[:, b0:b0 + 12] * inv[:, h:h + 1])
        optn.append(jnp.concatenate(blk, axis=1) - t[c])
    optl = [R[0][i] * optn[0] + R[1][i] * optn[1] + R[2][i] * optn[2]
            for i in range(3)]
    parts.extend(optl)
    parts.append(jnp.sqrt(optl[0] ** 2 + optl[1] ** 2 + optl[2] ** 2 + 1e-8))
    for h in range(H):
        b0 = OPAIR_OFF + h * 128
        parts.append(acc[:, b0:b0 + 128] * inv[:, h:h + 1])
    cat = jnp.concatenate(parts, axis=1)
    u = jnp.dot(cat, wo_ref[...], preferred_element_type=jnp.float32) + bo_ref[...]
    u = u * rm_ref[...]
    sp = s_ref[...] + u
    m = sp.mean(-1, keepdims=True)
    v = ((sp - m) ** 2).mean(-1, keepdims=True)
    out_ref[...] = (sp - m) / jnp.sqrt(v + 1e-5) * g_ref[...] + b_ref[...]


def _post(acc, s, rg, rmask2d, p, pre):
    n = s.shape[0]
    bn = 1000 if n % 1000 == 0 else 16
    full = lambda shape: pl.BlockSpec(shape, lambda i: (0, 0))
    return pl.pallas_call(
        _post_body,
        grid=(n // bn,),
        in_specs=[pl.BlockSpec((bn, ACC_W), lambda i: (i, 0)),
                  pl.BlockSpec((bn, 256), lambda i: (i, 0)),
                  pl.BlockSpec((bn, 7), lambda i: (i, 0)),
                  full((1664, 256)), full((1, 256)), full((1, 256)),
                  full((1, 256)),
                  pl.BlockSpec((bn, 1), lambda i: (i, 0))],
        out_specs=pl.BlockSpec((bn, 256), lambda i: (i, 0)),
        out_shape=jax.ShapeDtypeStruct((n, 256), jnp.float32),
    )(acc, s, rg, p[pre + 'Wo_p'], p[pre + 'bo'].reshape(1, -1),
      p['g1'].reshape(1, -1), p['b1'].reshape(1, -1), rmask2d)


def _final_body(s_ref, rg_ref, lf_ref, rm_ref, nm_ref, w1_ref, b1_ref, w2_ref,
                b2_ref, w3_ref, b3_ref, g_ref, bg_ref, wbb_ref, bbb_ref,
                wlu_ref, blu_ref, s_out, rig_out, lt_out):
    s = s_ref[...]
    x = jnp.maximum(jnp.dot(s, w1_ref[...], preferred_element_type=jnp.float32)
                    + b1_ref[...], 0.0)
    x = jnp.maximum(jnp.dot(x, w2_ref[...], preferred_element_type=jnp.float32)
                    + b2_ref[...], 0.0)
    x = jnp.dot(x, w3_ref[...], preferred_element_type=jnp.float32) + b3_ref[...]
    sp = s + x
    m = sp.mean(-1, keepdims=True)
    v = ((sp - m) ** 2).mean(-1, keepdims=True)
    sn = (sp - m) / jnp.sqrt(v + 1e-5) * g_ref[...] + bg_ref[...]
    rm = rm_ref[...]
    nm = nm_ref[...]
    sn = sn * rm
    s_out[...] = sn
    ub = jnp.dot(sn * nm, wbb_ref[...], preferred_element_type=jnp.float32) \
        + bbb_ref[...]
    ub = ub * nm
    R, t, qn = _frames(rg_ref[...])
    w0, x0, y0, z0 = qn[:, 0:1], qn[:, 1:2], qn[:, 2:3], qn[:, 3:4]
    b0, b1c, b2c = ub[:, 0:1], ub[:, 1:2], ub[:, 2:3]
    qw = w0 - x0 * b0 - y0 * b1c - z0 * b2c
    qx = w0 * b0 + x0 + y0 * b2c - z0 * b1c
    qy = w0 * b1c - x0 * b2c + y0 + z0 * b0
    qz = w0 * b2c + x0 * b1c - y0 * b0 + z0
    qnr = jnp.sqrt(qw * qw + qx * qx + qy * qy + qz * qz)
    u3 = (ub[:, 3:4], ub[:, 4:5], ub[:, 5:6])
    tn = [t[i] + R[i][0] * u3[0] + R[i][1] * u3[1] + R[i][2] * u3[2]
          for i in range(3)]
    rig_out[...] = jnp.concatenate(
        [qw / qnr, qx / qnr, qy / qnr, qz / qnr, tn[0], tn[1], tn[2]], axis=1)
    lt_out[...] = lf_ref[...] + jnp.dot(
        sn, wlu_ref[...], preferred_element_type=jnp.float32) + blu_ref[...]


def _final(s, rg, lf, rmask2d, nmask2d, p):
    n = s.shape[0]
    bn = 1000 if n % 1000 == 0 else 16
    full = lambda shape: pl.BlockSpec(shape, lambda i: (0, 0))
    row = lambda wdt: pl.BlockSpec((bn, wdt), lambda i: (i, 0))
    return pl.pallas_call(
        _final_body,
        grid=(n // bn,),
        in_specs=[row(256), row(7), row(128), row(1), row(1),
                  full((256, 256)), full((1, 256)), full((256, 256)),
                  full((1, 256)), full((256, 256)), full((1, 256)),
                  full((1, 256)), full((1, 256)), full((256, 6)), full((1, 6)),
                  full((256, 128)), full((1, 128))],
        out_specs=[row(256), row(7), row(128)],
        out_shape=[jax.ShapeDtypeStruct((n, 256), jnp.float32),
                   jax.ShapeDtypeStruct((n, 7), jnp.float32),
                   jax.ShapeDtypeStruct((n, 128), jnp.float32)],
    )(s, rg, lf, rmask2d, nmask2d, p['Wt1'], p['bt1'].reshape(1, -1),
      p['Wt2'], p['bt2'].reshape(1, -1), p['Wt3'], p['bt3'].reshape(1, -1),
      p['gt'].reshape(1, -1), p['bgt'].reshape(1, -1), p['Wbb'],
      p['bbb'].reshape(1, -1), p['Wlu'], p['blu'].reshape(1, -1))


def _perm3(npnts):
    idx = []
    for c in range(3):
        for hp in range(npnts):
            idx.append(hp * 3 + c)
    return np.array(idx, np.int32)


def _wo_perm():
    idx = list(range(256))
    for c in range(3):
        for h in range(H):
            for pv in range(PV):
                idx.append(256 + (h * PV + pv) * 3 + c)
    idx.extend(range(544, 1664))
    return np.array(idx, np.int32)


def _prep_params(p):
    q = dict(p)
    pqk_perm = _perm3(H * PQK)
    pv_perm = _perm3(H * PV)
    wo_perm = _wo_perm()
    for pre in ('sp_', 'sq_'):
        q[pre + 'Wqp_p'] = p[pre + 'Wqp'][:, pqk_perm]
        q[pre + 'Wkp_p'] = p[pre + 'Wkp'][:, pqk_perm]
        q[pre + 'Wvp_p'] = p[pre + 'Wvp'][:, pv_perm]
        q[pre + 'Wo_p'] = p[pre + 'Wo'][wo_perm, :]
    return q


def _ipa_stage(s, rg, etab, dstx, srcx, e2, n, rmask2d, p, pre):
    nb = n // NR
    nbp = ((nb + 1 + 15) // 16) * 16
    lp = e2 + NW * (nb + 1)
    edummy = e2
    dtab, stab, hwc = _tables(s, rg, p, pre)
    hist = _hist(dstx, e2, nbp)
    start, off, pc, cnt = _prefix(hist, nbp)
    off2d = jnp.broadcast_to(off[0][:, None], (nbp, 16))
    pc2d = jnp.broadcast_to(pc[0][:, None], (nbp, 16))
    bid, bsrc, bdst = _place(dstx, srcx, start, off, pc, cnt, e2, nb, nbp,
                             lp, edummy)
    eye16 = jnp.eye(16, dtype=jnp.float32)
    acc = _attn(dtab, stab, etab, bid, bsrc, bdst, off2d, pc2d, hwc, eye16,
                n, nb, nbp)
    return _post(acc, s, rg, rmask2d, p, pre)


def kernel(node_features, latent_features, rigids7, edge_features,
           seq_edge_features, edge_index, seq_edge_index, res_mask,
           noising_mask, params):
    p = _prep_params(params)
    n = node_features.shape[0]
    e = edge_features.shape[0]
    es = seq_edge_features.shape[0]
    e2a = ((e + 255) // 256) * 256
    e2b = ((es + 255) // 256) * 256

    rg = rigids7
    rmask2d = res_mask.reshape(-1, 1)
    nmask2d = noising_mask.reshape(-1, 1)

    def extend(ei, e_sz, e2_sz):
        src, dst = ei[0], ei[1]
        pads = e2_sz - e_sz
        srcx = jnp.concatenate([src, jnp.zeros((pads + 16,), jnp.int32)])
        dstx = jnp.concatenate([dst, jnp.full((pads,), n, jnp.int32),
                                jnp.full((16,), -16, jnp.int32)])
        return srcx, dstx

    srcx1, dstx1 = extend(edge_index, e, e2a)
    srcx2, dstx2 = extend(seq_edge_index, es, e2b)

    etab1 = _edge_mlp(edge_features, p)
    sefp = jnp.concatenate(
        [seq_edge_features, jnp.zeros((e2b - es, CZ), jnp.float32)])
    etab2 = _edge_tab2(sefp, p)

    nflf = jnp.concatenate([node_features, latent_features], axis=1)
    s0 = _s0(nflf, p)

    s1 = _ipa_stage(s0, rg, etab1, dstx1, srcx1, e2a, n, rmask2d, p, 'sp_')
    s2 = _ipa_stage(s1, rg, etab2, dstx2, srcx2, e2b, n, rmask2d, p, 'sq_')

    sfin, rig, lt = _final(s2, rg, latent_features, rmask2d, nmask2d, p)
    return sfin, rig, lt


# speedup vs baseline: 12.4001x; 1.0578x over previous
import functools
import numpy as np

import jax
import jax.numpy as jnp
from jax import lax
from jax.experimental import pallas as pl
from jax.experimental.pallas import tpu as pltpu
from jax.experimental.pallas import tpu_sc as plsc

H = 8
D = 32
PQK = 8
PV = 12
CZ = 128

NR = 16
NW = 32
FC = 8

RSQK = 1.0 / np.sqrt(3.0 * D)
RS3 = 1.0 / np.sqrt(3.0)
CPT = 0.5 * np.sqrt(1.0 / (3.0 * PQK * 4.5))

ACC_W = 1696
O_OFF = 16
OPT_OFF = 272
OPAIR_OFF = 656
SV_OFF = 512
SVP_OFF = 768
SRC_W = 1152
ET_W = 256

@functools.lru_cache(maxsize=1)
def _sc_mesh():
    return plsc.VectorSubcoreMesh(core_axis_name="c", subcore_axis_name="s",
                                  num_cores=2, num_subcores=16)


def _wid():
    return lax.axis_index("s") * 2 + lax.axis_index("c")


def _pick_cb(ew):
    for cb in range(128, 7, -8):
        if ew % cb == 0:
            return cb
    raise ValueError(ew)


def _vgather(vec, idx):
    dn = lax.GatherDimensionNumbers(offset_dims=(), collapsed_slice_dims=(0,),
                                    start_index_map=(0,))
    return lax.gather(vec, idx[:, None], dimension_numbers=dn, slice_sizes=(1,),
                      mode=lax.GatherScatterMode.PROMISE_IN_BOUNDS)


def _frames(rg):
    qq = rg[:, 0:4]
    nrm = jnp.sqrt(jnp.sum(qq * qq, -1, keepdims=True))
    qqn = qq / nrm
    w, x, y, z = qqn[:, 0:1], qqn[:, 1:2], qqn[:, 2:3], qqn[:, 3:4]
    R = ((1 - 2 * (y * y + z * z), 2 * (x * y - w * z), 2 * (x * z + w * y)),
        (2 * (x * y + w * z), 1 - 2 * (x * x + z * z), 2 * (y * z - w * x)),
        (2 * (x * z - w * y), 2 * (y * z + w * x), 1 - 2 * (x * x + y * y)))
    t = (rg[:, 4:5], rg[:, 5:6], rg[:, 6:7])
    return R, t, qqn


def _edge_mlp_body(x_ref, w0_ref, b0_ref, w1_ref, b1_ref, w2_ref, b2_ref,
                   g_ref, bg_ref, wb_ref, out_ref):
    x = x_ref[...]
    h = jnp.maximum(jnp.dot(x, w0_ref[...], preferred_element_type=jnp.float32)
                    + b0_ref[...], 0.0)
    h = jnp.maximum(jnp.dot(h, w1_ref[...], preferred_element_type=jnp.float32)
                    + b1_ref[...], 0.0)
    zz = jnp.dot(h, w2_ref[...], preferred_element_type=jnp.float32) + b2_ref[...]
    m = zz.mean(-1, keepdims=True)
    v = ((zz - m) ** 2).mean(-1, keepdims=True)
    zn = (zz - m) / jnp.sqrt(v + 1e-5) * g_ref[...] + bg_ref[...]
    out_ref[:, 0:CZ] = zn
    out_ref[:, CZ:CZ + H] = jnp.dot(zn, wb_ref[...],
                                    preferred_element_type=jnp.float32)
    out_ref[:, CZ + H:ET_W] = jnp.zeros_like(out_ref[:, CZ + H:ET_W])


def _edge_mlp(ef, p):
    e = ef.shape[0]
    be = 1600 if e % 1600 == 0 else 128
    nblk = e // be
    full = lambda shape: pl.BlockSpec(shape, lambda i: (0, 0))
    inrow = pl.BlockSpec((be, CZ), lambda i: (jnp.minimum(i, nblk - 1), 0))
    return pl.pallas_call(
        _edge_mlp_body,
        grid=(nblk + 1,),
        in_specs=[inrow, full((CZ, CZ)), full((1, CZ)), full((CZ, CZ)),
                  full((1, CZ)), full((CZ, CZ)), full((1, CZ)), full((1, CZ)),
                  full((1, CZ)), full((CZ, H))],
        out_specs=pl.BlockSpec((be, ET_W), lambda i: (i, 0)),
        out_shape=jax.ShapeDtypeStruct(((nblk + 1) * be, ET_W), jnp.float32),
    )(ef, p['We0'], p['be0'].reshape(1, -1), p['We1'], p['be1'].reshape(1, -1),
      p['We2'], p['be2'].reshape(1, -1), p['ge'].reshape(1, -1),
      p['bge'].reshape(1, -1), p['sp_Wb'])


def _edge_tab2_body(x_ref, wb_ref, out_ref):
    x = x_ref[...]
    out_ref[:, 0:CZ] = x
    out_ref[:, CZ:CZ + H] = jnp.dot(x, wb_ref[...],
                                    preferred_element_type=jnp.float32)
    out_ref[:, CZ + H:ET_W] = jnp.zeros_like(out_ref[:, CZ + H:ET_W])


def _edge_tab2(sefp, p):
    e = sefp.shape[0]
    be = 1280 if e % 1280 == 0 else 128
    nblk = e // be
    return pl.pallas_call(
        _edge_tab2_body,
        grid=(nblk + 1,),
        in_specs=[pl.BlockSpec((be, CZ), lambda i: (jnp.minimum(i, nblk - 1), 0)),
                  pl.BlockSpec((CZ, H), lambda i: (0, 0))],
        out_specs=pl.BlockSpec((be, ET_W), lambda i: (i, 0)),
        out_shape=jax.ShapeDtypeStruct(((nblk + 1) * be, ET_W), jnp.float32),
    )(sefp, p['sq_Wb'])


def _s0_body(x_ref, w_ref, b_ref, o_ref):
    o_ref[...] = jnp.dot(x_ref[...], w_ref[...],
                         preferred_element_type=jnp.float32) + b_ref[...]


def _s0(nflf, p):
    n = nflf.shape[0]
    bn = 1000 if n % 1000 == 0 else 16
    cin = nflf.shape[1]
    return pl.pallas_call(
        _s0_body,
        grid=(n // bn,),
        in_specs=[pl.BlockSpec((bn, cin), lambda i: (i, 0)),
                  pl.BlockSpec((cin, 256), lambda i: (0, 0)),
                  pl.BlockSpec((1, 256), lambda i: (0, 0))],
        out_specs=pl.BlockSpec((bn, 256), lambda i: (i, 0)),
        out_shape=jax.ShapeDtypeStruct((n, 256), jnp.float32),
    )(nflf, p['Wnu'], p['bnu'].reshape(1, -1))


def _tab_body(s_ref, rg_ref, wq_ref, wk_ref, wv_ref, wqp_ref, wkp_ref, wvp_ref,
              hw_ref, dst_ref, src_ref, hwc_ref):
    s = s_ref[...]
    R, t, _ = _frames(rg_ref[...])
    q = jnp.dot(s, wq_ref[...], preferred_element_type=jnp.float32)
    k = jnp.dot(s, wk_ref[...], preferred_element_type=jnp.float32)
    v = jnp.dot(s, wv_ref[...], preferred_element_type=jnp.float32)
    qp3 = jnp.dot(s, wqp_ref[...], preferred_element_type=jnp.float32)
    kp3 = jnp.dot(s, wkp_ref[...], preferred_element_type=jnp.float32)
    vp3 = jnp.dot(s, wvp_ref[...], preferred_element_type=jnp.float32)

    def rot(p3, width):
        px = p3[:, 0 * width:1 * width]
        py = p3[:, 1 * width:2 * width]
        pz = p3[:, 2 * width:3 * width]
        return tuple(R[i][0] * px + R[i][1] * py + R[i][2] * pz + t[i]
                     for i in range(3))

    qpg = rot(qp3, H * PQK)
    kpg = rot(kp3, H * PQK)
    vpg = rot(vp3, H * PV)
    zpad = jnp.zeros_like(s[:, 0:8])
    for h in range(H):
        b = h * 64
        dst_ref[:, b:b + 32] = q[:, h * 32:h * 32 + 32]
        src_ref[:, b:b + 32] = k[:, h * 32:h * 32 + 32]
        for c in range(3):
            dst_ref[:, b + 32 + c * 8:b + 40 + c * 8] = qpg[c][:, h * 8:h * 8 + 8]
            src_ref[:, b + 32 + c * 8:b + 40 + c * 8] = kpg[c][:, h * 8:h * 8 + 8]
        dst_ref[:, b + 56:b + 64] = zpad
        src_ref[:, b + 56:b + 64] = zpad
    src_ref[:, SV_OFF:SV_OFF + 256] = v
    zpad4 = jnp.zeros_like(s[:, 0:4])
    for c in range(3):
        for h in range(H):
            b0 = SVP_OFF + c * 128 + h * 16
            src_ref[:, b0:b0 + 12] = vpg[c][:, h * 12:(h + 1) * 12]
            src_ref[:, b0 + 12:b0 + 16] = zpad4
    hw = hw_ref[...]
    ch = jnp.log1p(jnp.exp(hw)) * CPT
    hwc_ref[...] = jnp.concatenate(
        [ch, jnp.zeros((1, 128 - H), jnp.float32)], axis=1)


def _tables(s, rg, p, pre):
    n = s.shape[0]
    bn = 1000 if n % 1000 == 0 else 16
    full = lambda shape: pl.BlockSpec(shape, lambda i: (0, 0))
    return pl.pallas_call(
        _tab_body,
        grid=(n // bn,),
        in_specs=[pl.BlockSpec((bn, 256), lambda i: (i, 0)),
                  pl.BlockSpec((bn, 7), lambda i: (i, 0)),
                  full((256, 256)), full((256, 256)), full((256, 256)),
                  full((256, 192)), full((256, 192)), full((256, 288)),
                  full((1, H))],
        out_specs=[pl.BlockSpec((bn, 512), lambda i: (i, 0)),
                   pl.BlockSpec((bn, SRC_W), lambda i: (i, 0)),
                   pl.BlockSpec((1, 128), lambda i: (0, 0))],
        out_shape=[jax.ShapeDtypeStruct((n, 512), jnp.float32),
                   jax.ShapeDtypeStruct((n, SRC_W), jnp.float32),
                   jax.ShapeDtypeStruct((1, 128), jnp.float32)],
    )(s, rg, p[pre + 'Wq'], p[pre + 'Wk'], p[pre + 'Wv'], p[pre + 'Wqp_p'],
      p[pre + 'Wkp_p'], p[pre + 'Wvp_p'], p[pre + 'head_w'].reshape(1, -1))


def _hist(dstx, e2, nbp):
    ew = e2 // NW
    cb = _pick_cb(ew)

    def body(dst_hbm, hist_hbm, dbuf, histv, i32z16):
        w = _wid()
        zv = jnp.zeros((16,), jnp.int32)
        for j in range(nbp // 16):
            histv[pl.ds(j * 16, 16)] = zv

        def chunk(c, _):
            pltpu.sync_copy(dst_hbm.at[pl.ds(w * ew + c * cb, cb)], dbuf)

            def per(j, _):
                b = dbuf[pl.ds(j, 1)][0] >> 4
                old = histv[pl.ds(b, 1)][0]
                histv[pl.ds(b, 1)] = jnp.full((1,), old + 1, jnp.int32)
                return 0
            lax.fori_loop(0, cb, per, 0)
            return 0
        lax.fori_loop(0, ew // cb, chunk, 0)
        pltpu.sync_copy(histv, hist_hbm.at[w])

    return pl.kernel(
        body, out_type=jax.ShapeDtypeStruct((NW, nbp), jnp.int32),
        mesh=_sc_mesh(),
        scratch_types=[pltpu.VMEM((cb,), jnp.int32),
                       pltpu.VMEM((nbp,), jnp.int32),
                       pltpu.VMEM((16,), jnp.int32)],
    )(dstx)


def _prefix_body(h_ref, start_ref, off_ref, pc_ref, cnt_ref):
    h = h_ref[...].astype(jnp.float32)
    nbp = h.shape[1]
    wi = lax.broadcasted_iota(jnp.int32, (NW, NW), 0)
    wj = lax.broadcasted_iota(jnp.int32, (NW, NW), 1)
    mlow = (wj < wi).astype(jnp.float32)
    below = jnp.dot(mlow, h, preferred_element_type=jnp.float32)
    cnt = jnp.sum(h, axis=0, keepdims=True)
    pc = jnp.floor((cnt + 31.0) * (1.0 / 32.0)) * 32.0
    bi = lax.broadcasted_iota(jnp.int32, (nbp, nbp), 0)
    bj = lax.broadcasted_iota(jnp.int32, (nbp, nbp), 1)
    mb = (bi < bj).astype(jnp.float32)
    off = jnp.dot(pc, mb, preferred_element_type=jnp.float32)
    start_ref[...] = jnp.round(below + off).astype(jnp.int32)
    ones8 = jnp.ones((8, 1), jnp.float32)
    off_ref[...] = jnp.round(ones8 * off).astype(jnp.int32)
    pc_ref[...] = jnp.round(ones8 * pc).astype(jnp.int32)
    cnt_ref[...] = jnp.round(ones8 * cnt).astype(jnp.int32)


def _prefix(hist, nbp):
    full = lambda shape: pl.BlockSpec(shape, lambda: (0, 0))
    return pl.pallas_call(
        _prefix_body,
        in_specs=[full((NW, nbp))],
        out_specs=[full((NW, nbp)), full((8, nbp)), full((8, nbp)),
                   full((8, nbp))],
        out_shape=[jax.ShapeDtypeStruct((NW, nbp), jnp.int32),
                   jax.ShapeDtypeStruct((8, nbp), jnp.int32),
                   jax.ShapeDtypeStruct((8, nbp), jnp.int32),
                   jax.ShapeDtypeStruct((8, nbp), jnp.int32)],
    )(hist)


def _place(dstx, srcx, start, off, pc, cnt, e2, nb, nbp, lp, edummy):
    ew = e2 // NW
    cb = _pick_cb(ew)

    def body(dst_hbm, src_hbm, start_hbm, off_hbm, pc_hbm, cnt_hbm,
             bid_hbm, bsrc_hbm, bdst_hbm,
             dbuf, sbuf, curv, posb, idb, offv, pcv, cntv, padpos, sent, sem):
        w = _wid()
        pltpu.sync_copy(start_hbm.at[w], curv)
        pltpu.sync_copy(off_hbm.at[0], offv)
        pltpu.sync_copy(pc_hbm.at[0], pcv)
        pltpu.sync_copy(cnt_hbm.at[0], cntv)
        sent[0, pl.ds(0, 16)] = jnp.full((16,), edummy, jnp.int32)
        sent[1, pl.ds(0, 16)] = jnp.zeros((16,), jnp.int32)
        sent[2, pl.ds(0, 16)] = jnp.full((16,), -16, jnp.int32)

        def chunk(c, _):
            pltpu.sync_copy(dst_hbm.at[pl.ds(w * ew + c * cb, cb)], dbuf)
            pltpu.sync_copy(src_hbm.at[pl.ds(w * ew + c * cb, cb)], sbuf)

            def per(j, _):
                b = dbuf[pl.ds(j, 1)][0] >> 4
                pos = curv[pl.ds(b, 1)][0]
                curv[pl.ds(b, 1)] = jnp.full((1,), pos + 1, jnp.int32)
                posb[pl.ds(j, 1)] = jnp.full((1,), pos, jnp.int32)
                idb[pl.ds(j, 1)] = jnp.full((1,), w * ew + c * cb + j,
                                            jnp.int32)
                return 0
            lax.fori_loop(0, cb, per, 0)
            pltpu.async_copy(idb, bid_hbm.at[posb], sem).wait()
            pltpu.async_copy(sbuf, bsrc_hbm.at[posb], sem).wait()
            pltpu.async_copy(dbuf, bdst_hbm.at[posb], sem).wait()
            return 0
        lax.fori_loop(0, ew // cb, chunk, 0)

        trips = jnp.maximum((nb - w) // NW + 1, 0)

        def padfill(i, _):
            b = w + i * NW
            pcb = pcv[pl.ds(b, 1)][0]
            cntb = cntv[pl.ds(b, 1)][0]
            offb = offv[pl.ds(b, 1)][0]
            pad = pcb - cntb
            base = offb + cntb
            last = offb + pcb - 1

            @pl.when(pad > 0)
            def _():
                for r in range(2):
                    for l in range(16):
                        padpos[r, pl.ds(l, 1)] = jnp.full(
                            (1,), jnp.minimum(base + r * 16 + l, last),
                            jnp.int32)
                for r in range(2):
                    pltpu.async_copy(sent.at[0], bid_hbm.at[padpos.at[r]], sem).wait()
                    pltpu.async_copy(sent.at[1], bsrc_hbm.at[padpos.at[r]], sem).wait()
                    pltpu.async_copy(sent.at[2], bdst_hbm.at[padpos.at[r]], sem).wait()
            return 0
        lax.fori_loop(0, trips, padfill, 0)

    return pl.kernel(
        body,
        out_type=[jax.ShapeDtypeStruct((lp,), jnp.int32),
                  jax.ShapeDtypeStruct((lp,), jnp.int32),
                  jax.ShapeDtypeStruct((lp,), jnp.int32)],
        mesh=_sc_mesh(),
        scratch_types=[pltpu.VMEM((cb,), jnp.int32),
                       pltpu.VMEM((cb,), jnp.int32),
                       pltpu.VMEM((nbp,), jnp.int32),
                       pltpu.VMEM((cb,), jnp.int32),
                       pltpu.VMEM((cb,), jnp.int32),
                       pltpu.VMEM((nbp,), jnp.int32),
                       pltpu.VMEM((nbp,), jnp.int32),
                       pltpu.VMEM((nbp,), jnp.int32),
                       pltpu.VMEM((2, 16), jnp.int32),
                       pltpu.VMEM((3, 16), jnp.int32),
                       pltpu.SemaphoreType.DMA],
    )(dstx, srcx, start, off, pc, cnt)


def _attn(dtab, stab, etab, bid, bsrc, bdst, off2d, pc2d, hwc, eye16,
          n, nb, nbp):
    def body(dtab_hbm, stab_hbm, etab_hbm, bid_hbm, bsrc_hbm, bdst_hbm,
             off_hbm, pc_hbm, hwc_hbm, eye_hbm, acc_hbm,
             hwv, dtabv, accv, obuf, pbuf,
             srcvA, idsvA, dstvA, srowsA, erowsA,
             srcvB, idsvB, dstvB, srowsB, erowsB,
             ohv, semA, semB):
        w = _wid()
        pltpu.sync_copy(hwc_hbm.at[0], hwv)
        pltpu.sync_copy(eye_hbm, ohv)
        onehots = [ohv[hh, pl.ds(0, 16)] for hh in range(H)]
        hv = hwv[pl.ds(0, 16)]
        z16 = jnp.zeros((16,), jnp.float32)
        trips = ((nb - 1 - w) >> 5) + 1
        slots = ((srcvA, idsvA, dstvA, srowsA, erowsA, semA),
                 (srcvB, idsvB, dstvB, srowsB, erowsB, semB))

        def load_lin(slot, base):
            srcv, idsv, dstv, srows, erows, sem = slot
            pltpu.sync_copy(bid_hbm.at[pl.ds(base, FC)], idsv)
            pltpu.sync_copy(bsrc_hbm.at[pl.ds(base, FC)], srcv)
            pltpu.sync_copy(bdst_hbm.at[pl.ds(base, FC)],
                            dstv.at[pl.ds(0, FC)])
            pltpu.async_copy(stab_hbm.at[srcv], srows, sem)
            pltpu.async_copy(etab_hbm.at[idsv], erows, sem)

        def wait_slot(slot):
            srcv, idsv, dstv, srows, erows, sem = slot
            pltpu.make_async_copy(stab_hbm.at[srcv], srows, sem).wait()
            pltpu.make_async_copy(etab_hbm.at[idsv], erows, sem).wait()

        def compute(slot, b, nb0):
            srcv, idsv, dstv, srows, erows, sem = slot
            dsts = dstv[...]
            for j in range(FC):
                d = dsts[j]
                dl = jnp.where((d >> 4) == b, d - nb0, NR)
                dlr = jnp.minimum(dl, NR - 1)
                bbv = erows[j, pl.ds(CZ, 16)]
                avec = z16
                for h in range(H):
                    cb0 = h * 64
                    qv0 = dtabv[dlr, pl.ds(cb0, 16)]
                    qv1 = dtabv[dlr, pl.ds(cb0 + 16, 16)]
                    qv2 = dtabv[dlr, pl.ds(cb0 + 32, 16)]
                    qv3 = dtabv[dlr, pl.ds(cb0 + 48, 16)]
                    kv0 = srows[j, pl.ds(cb0, 16)]
                    kv1 = srows[j, pl.ds(cb0 + 16, 16)]
                    kv2 = srows[j, pl.ds(cb0 + 32, 16)]
                    kv3 = srows[j, pl.ds(cb0 + 48, 16)]
                    dq2 = qv2 - kv2
                    dq3 = qv3 - kv3
                    comb = ((qv0 * kv0 + qv1 * kv1) * RSQK
                            - hv[h] * (dq2 * dq2 + dq3 * dq3))
                    a = jnp.sum(comb) + bbv[h] * RS3
                    avec = avec + a * onehots[h]
                vea = jnp.exp(avec)
                accv[dl, pl.ds(0, 16)] = accv[dl, pl.ds(0, 16)] + vea
                zv = [erows[j, pl.ds(r * 16, 16)] for r in range(8)]
                for h in range(H):
                    eb = jnp.full((16,), vea[h], jnp.float32)
                    for r in range(2):
                        co = O_OFF + h * 32 + r * 16
                        si = SV_OFF + h * 32 + r * 16
                        accv[dl, pl.ds(co, 16)] = (
                            accv[dl, pl.ds(co, 16)]
                            + eb * srows[j, pl.ds(si, 16)])
                    for c in range(3):
                        co = OPT_OFF + c * 128 + h * 16
                        si = SVP_OFF + c * 128 + h * 16
                        accv[dl, pl.ds(co, 16)] = (
                            accv[dl, pl.ds(co, 16)]
                            + eb * srows[j, pl.ds(si, 16)])
                    for r in range(8):
                        cp = OPAIR_OFF + h * 128 + r * 16
                        accv[dl, pl.ds(cp, 16)] = (
                            accv[dl, pl.ds(cp, 16)] + eb * zv[r])

        def bucket(i, _):
            b = w + i * NW
            nb0 = b * NR

            def zrow(r, _2):
                for cix in range(ACC_W // 16):
                    accv[r, pl.ds(cix * 16, 16)] = z16
                return 0
            lax.fori_loop(0, NR + 1, zrow, 0)
            pltpu.sync_copy(dtab_hbm.at[pl.ds(nb0, NR)], dtabv)
            pltpu.sync_copy(off_hbm.at[b], obuf)
            pltpu.sync_copy(pc_hbm.at[b], pbuf)
            o0 = obuf[...][0]
            pcs = pbuf[...][0]
            lastb = pl.multiple_of(o0 + pcs - FC, 8)

            @pl.when(pcs > 0)
            def _():
                load_lin(slots[0], pl.multiple_of(o0, 8))

                def chunk2(ci2, _2):
                    cA = ci2 * 2
                    baseB = pl.multiple_of(
                        jnp.minimum(o0 + (cA + 1) * FC, lastb), 8)
                    wait_slot(slots[0])
                    load_lin(slots[1], baseB)
                    compute(slots[0], b, nb0)
                    baseA2 = pl.multiple_of(
                        jnp.minimum(o0 + (cA + 2) * FC, lastb), 8)
                    wait_slot(slots[1])
                    load_lin(slots[0], baseA2)
                    compute(slots[1], b, nb0)
                    return 0
                lax.fori_loop(0, pcs >> 4, chunk2, 0)
                wait_slot(slots[0])
            pltpu.sync_copy(accv.at[pl.ds(0, NR)], acc_hbm.at[pl.ds(nb0, NR)])
            return 0
        lax.fori_loop(0, trips, bucket, 0)

    buf = lambda: [pltpu.VMEM((FC,), jnp.int32), pltpu.VMEM((FC,), jnp.int32),
                   pltpu.VMEM((16,), jnp.int32),
                   pltpu.VMEM((FC, SRC_W), jnp.float32),
                   pltpu.VMEM((FC, ET_W), jnp.float32)]
    return pl.kernel(
        body, out_type=jax.ShapeDtypeStruct((n, ACC_W), jnp.float32),
        mesh=_sc_mesh(),
        compiler_params=pltpu.CompilerParams(needs_layout_passes=False),
        scratch_types=[pltpu.VMEM((128,), jnp.float32),
                       pltpu.VMEM((NR, 512), jnp.float32),
                       pltpu.VMEM((NR + 1, ACC_W), jnp.float32),
                       pltpu.VMEM((16,), jnp.int32),
                       pltpu.VMEM((16,), jnp.int32)]
                      + buf() + buf()
                      + [pltpu.VMEM((16, 16), jnp.float32),
                         pltpu.SemaphoreType.DMA,
                         pltpu.SemaphoreType.DMA],
    )(dtab, stab, etab, bid, bsrc, bdst, off2d, pc2d, hwc, eye16)


def _post_body(acc_ref, s_ref, rg_ref, wo_ref, bo_ref, g_ref, b_ref, rm_ref,
               out_ref):
    acc = acc_ref[...]
    R, t, _ = _frames(rg_ref[...])
    den = acc[:, 0:H]
    inv = 1.0 / (den + 1e-9)
    parts = []
    for h in range(H):
        parts.append(acc[:, O_OFF + h * 32:O_OFF + (h + 1) * 32]
                     * inv[:, h:h + 1])
    optn = []
    for c in range(3):
        blk = []
        for h in range(H):
            b0 = OPT_OFF + c * 128 + h * 16
            blk.append(acc[:, b0:b0 + 12] * inv[:, h:h + 1])
        optn.append(jnp.concatenate(blk, axis=1) - t[c])
    optl = [R[0][i] * optn[0] + R[1][i] * optn[1] + R[2][i] * optn[2]
            for i in range(3)]
    parts.extend(optl)
    parts.append(jnp.sqrt(optl[0] ** 2 + optl[1] ** 2 + optl[2] ** 2 + 1e-8))
    for h in range(H):
        b0 = OPAIR_OFF + h * 128
        parts.append(acc[:, b0:b0 + 128] * inv[:, h:h + 1])
    cat = jnp.concatenate(parts, axis=1)
    u = jnp.dot(cat, wo_ref[...], preferred_element_type=jnp.float32) + bo_ref[...]
    u = u * rm_ref[...]
    sp = s_ref[...] + u
    m = sp.mean(-1, keepdims=True)
    v = ((sp - m) ** 2).mean(-1, keepdims=True)
    out_ref[...] = (sp - m) / jnp.sqrt(v + 1e-5) * g_ref[...] + b_ref[...]


def _post(acc, s, rg, rmask2d, p, pre):
    n = s.shape[0]
    bn = 1000 if n % 1000 == 0 else 16
    full = lambda shape: pl.BlockSpec(shape, lambda i: (0, 0))
    return pl.pallas_call(
        _post_body,
        grid=(n // bn,),
        in_specs=[pl.BlockSpec((bn, ACC_W), lambda i: (i, 0)),
                  pl.BlockSpec((bn, 256), lambda i: (i, 0)),
                  pl.BlockSpec((bn, 7), lambda i: (i, 0)),
                  full((1664, 256)), full((1, 256)), full((1, 256)),
                  full((1, 256)),
                  pl.BlockSpec((bn, 1), lambda i: (i, 0))],
        out_specs=pl.BlockSpec((bn, 256), lambda i: (i, 0)),
        out_shape=jax.ShapeDtypeStruct((n, 256), jnp.float32),
    )(acc, s, rg, p[pre + 'Wo_p'], p[pre + 'bo'].reshape(1, -1),
      p['g1'].reshape(1, -1), p['b1'].reshape(1, -1), rmask2d)


def _final_body(s_ref, rg_ref, lf_ref, rm_ref, nm_ref, w1_ref, b1_ref, w2_ref,
                b2_ref, w3_ref, b3_ref, g_ref, bg_ref, wbb_ref, bbb_ref,
                wlu_ref, blu_ref, s_out, rig_out, lt_out):
    s = s_ref[...]
    x = jnp.maximum(jnp.dot(s, w1_ref[...], preferred_element_type=jnp.float32)
                    + b1_ref[...], 0.0)
    x = jnp.maximum(jnp.dot(x, w2_ref[...], preferred_element_type=jnp.float32)
                    + b2_ref[...], 0.0)
    x = jnp.dot(x, w3_ref[...], preferred_element_type=jnp.float32) + b3_ref[...]
    sp = s + x
    m = sp.mean(-1, keepdims=True)
    v = ((sp - m) ** 2).mean(-1, keepdims=True)
    sn = (sp - m) / jnp.sqrt(v + 1e-5) * g_ref[...] + bg_ref[...]
    rm = rm_ref[...]
    nm = nm_ref[...]
    sn = sn * rm
    s_out[...] = sn
    ub = jnp.dot(sn * nm, wbb_ref[...], preferred_element_type=jnp.float32) \
        + bbb_ref[...]
    ub = ub * nm
    R, t, qn = _frames(rg_ref[...])
    w0, x0, y0, z0 = qn[:, 0:1], qn[:, 1:2], qn[:, 2:3], qn[:, 3:4]
    b0, b1c, b2c = ub[:, 0:1], ub[:, 1:2], ub[:, 2:3]
    qw = w0 - x0 * b0 - y0 * b1c - z0 * b2c
    qx = w0 * b0 + x0 + y0 * b2c - z0 * b1c
    qy = w0 * b1c - x0 * b2c + y0 + z0 * b0
    qz = w0 * b2c + x0 * b1c - y0 * b0 + z0
    qnr = jnp.sqrt(qw * qw + qx * qx + qy * qy + qz * qz)
    u3 = (ub[:, 3:4], ub[:, 4:5], ub[:, 5:6])
    tn = [t[i] + R[i][0] * u3[0] + R[i][1] * u3[1] + R[i][2] * u3[2]
          for i in range(3)]
    rig_out[...] = jnp.concatenate(
        [qw / qnr, qx / qnr, qy / qnr, qz / qnr, tn[0], tn[1], tn[2]], axis=1)
    lt_out[...] = lf_ref[...] + jnp.dot(
        sn, wlu_ref[...], preferred_element_type=jnp.float32) + blu_ref[...]


def _final(s, rg, lf, rmask2d, nmask2d, p):
    n = s.shape[0]
    bn = 1000 if n % 1000 == 0 else 16
    full = lambda shape: pl.BlockSpec(shape, lambda i: (0, 0))
    row = lambda wdt: pl.BlockSpec((bn, wdt), lambda i: (i, 0))
    return pl.pallas_call(
        _final_body,
        grid=(n // bn,),
        in_specs=[row(256), row(7), row(128), row(1), row(1),
                  full((256, 256)), full((1, 256)), full((256, 256)),
                  full((1, 256)), full((256, 256)), full((1, 256)),
                  full((1, 256)), full((1, 256)), full((256, 6)), full((1, 6)),
                  full((256, 128)), full((1, 128))],
        out_specs=[row(256), row(7), row(128)],
        out_shape=[jax.ShapeDtypeStruct((n, 256), jnp.float32),
                   jax.ShapeDtypeStruct((n, 7), jnp.float32),
                   jax.ShapeDtypeStruct((n, 128), jnp.float32)],
    )(s, rg, lf, rmask2d, nmask2d, p['Wt1'], p['bt1'].reshape(1, -1),
      p['Wt2'], p['bt2'].reshape(1, -1), p['Wt3'], p['bt3'].reshape(1, -1),
      p['gt'].reshape(1, -1), p['bgt'].reshape(1, -1), p['Wbb'],
      p['bbb'].reshape(1, -1), p['Wlu'], p['blu'].reshape(1, -1))


def _perm3(npnts):
    idx = []
    for c in range(3):
        for hp in range(npnts):
            idx.append(hp * 3 + c)
    return np.array(idx, np.int32)


def _wo_perm():
    idx = list(range(256))
    for c in range(3):
        for h in range(H):
            for pv in range(PV):
                idx.append(256 + (h * PV + pv) * 3 + c)
    idx.extend(range(544, 1664))
    return np.array(idx, np.int32)


def _prep_params(p):
    q = dict(p)
    pqk_perm = _perm3(H * PQK)
    pv_perm = _perm3(H * PV)
    wo_perm = _wo_perm()
    for pre in ('sp_', 'sq_'):
        q[pre + 'Wqp_p'] = p[pre + 'Wqp'][:, pqk_perm]
        q[pre + 'Wkp_p'] = p[pre + 'Wkp'][:, pqk_perm]
        q[pre + 'Wvp_p'] = p[pre + 'Wvp'][:, pv_perm]
        q[pre + 'Wo_p'] = p[pre + 'Wo'][wo_perm, :]
    return q


def _ipa_stage(s, rg, etab, dstx, srcx, e2, n, rmask2d, p, pre):
    nb = n // NR
    nbp = ((nb + 1 + 15) // 16) * 16
    lp = e2 + NW * (nb + 1)
    edummy = e2
    dtab, stab, hwc = _tables(s, rg, p, pre)
    hist = _hist(dstx, e2, nbp)
    start, off, pc, cnt = _prefix(hist, nbp)
    off2d = jnp.broadcast_to(off[0][:, None], (nbp, 16))
    pc2d = jnp.broadcast_to(pc[0][:, None], (nbp, 16))
    bid, bsrc, bdst = _place(dstx, srcx, start, off, pc, cnt, e2, nb, nbp,
                             lp, edummy)
    eye16 = jnp.eye(16, dtype=jnp.float32)
    acc = _attn(dtab, stab, etab, bid, bsrc, bdst, off2d, pc2d, hwc, eye16,
                n, nb, nbp)
    return _post(acc, s, rg, rmask2d, p, pre)


def kernel(node_features, latent_features, rigids7, edge_features,
           seq_edge_features, edge_index, seq_edge_index, res_mask,
           noising_mask, params):
    p = _prep_params(params)
    n = node_features.shape[0]
    e = edge_features.shape[0]
    es = seq_edge_features.shape[0]
    e2a = ((e + 255) // 256) * 256
    e2b = ((es + 255) // 256) * 256

    rg = rigids7
    rmask2d = res_mask.reshape(-1, 1)
    nmask2d = noising_mask.reshape(-1, 1)

    def extend(ei, e_sz, e2_sz):
        src, dst = ei[0], ei[1]
        pads = e2_sz - e_sz
        srcx = jnp.concatenate([src, jnp.zeros((pads + 16,), jnp.int32)])
        dstx = jnp.concatenate([dst, jnp.full((pads,), n, jnp.int32),
                                jnp.full((16,), -16, jnp.int32)])
        return srcx, dstx

    srcx1, dstx1 = extend(edge_index, e, e2a)
    srcx2, dstx2 = extend(seq_edge_index, es, e2b)

    etab1 = _edge_mlp(edge_features, p)
    sefp = jnp.concatenate(
        [seq_edge_features, jnp.zeros((e2b - es, CZ), jnp.float32)])
    etab2 = _edge_tab2(sefp, p)

    nflf = jnp.concatenate([node_features, latent_features], axis=1)
    s0 = _s0(nflf, p)

    s1 = _ipa_stage(s0, rg, etab1, dstx1, srcx1, e2a, n, rmask2d, p, 'sp_')
    s2 = _ipa_stage(s1, rg, etab2, dstx2, srcx2, e2b, n, rmask2d, p, 'sq_')

    sfin, rig, lt = _final(s2, rg, latent_features, rmask2d, nmask2d, p)
    return sfin, rig, lt
